# baseline (speedup 1.0000x reference)
import functools

import jax
import jax.numpy as jnp
from jax import lax
from jax.experimental import pallas as pl
from jax.experimental.pallas import tpu as pltpu
from jax.experimental.pallas import tpu_sc as plsc

N = 10000
D = 128
E = 320000
NC = 2
NS = 16
NW = NC * NS
CHUNK = 128
NCHUNKS = E // CHUNK
ACC_ROWS = N + 112
RPT = ACC_ROWS // NS


def _tile_chunks(wid):
    base = jnp.where(wid < 2, 80 * wid, 160 + 78 * (wid - 2))
    count = jnp.where(wid < 2, 80, 78)
    return base, count


def _mesh():
    return plsc.VectorSubcoreMesh(
        core_axis_name="c", subcore_axis_name="s", num_cores=NC, num_subcores=NS
    )


@functools.partial(
    pl.kernel,
    out_type=jax.ShapeDtypeStruct((NW * ACC_ROWS,), jnp.float32),
    mesh=_mesh(),
    scratch_types=[
        pltpu.VMEM((ACC_ROWS,), jnp.float32),
        pltpu.VMEM((80 * CHUNK,), jnp.int32),
        pltpu.SemaphoreType.DMA,
    ],
    compiler_params=pltpu.CompilerParams(needs_layout_passes=False),
)
def _sc_deg(row_hbm, out_hbm, deg_v, idx_v, sem):
    c = lax.axis_index("c")
    s = lax.axis_index("s")
    wid = s * NC + c
    zeros16 = jnp.zeros((16,), jnp.float32)
    ones16 = jnp.ones((16,), jnp.float32)
    cbase, cnt = _tile_chunks(wid)

    idx_dma = pltpu.async_copy(
        row_hbm.at[pl.ds(cbase * CHUNK, 78 * CHUNK)],
        idx_v.at[pl.ds(0, 78 * CHUNK)],
        sem,
    )

    @pl.when(wid < 2)
    def _tail():
        pltpu.async_copy(
            row_hbm.at[pl.ds((cbase + 78) * CHUNK, 2 * CHUNK)],
            idx_v.at[pl.ds(78 * CHUNK, 2 * CHUNK)],
            sem,
        )

    @pl.loop(0, ACC_ROWS // 16)
    def _zero(i):
        deg_v[pl.ds(i * 16, 16)] = zeros16

    idx_dma.wait()

    @pl.when(wid < 2)
    def _tail_wait():
        pltpu.make_async_copy(
            row_hbm.at[pl.ds((cbase + 78) * CHUNK, 2 * CHUNK)],
            idx_v.at[pl.ds(78 * CHUNK, 2 * CHUNK)],
            sem,
        ).wait()

    @pl.loop(0, cnt * (CHUNK // 16))
    def _groups(i):
        idx16 = idx_v[pl.ds(i * 16, 16)]
        plsc.addupdate_scatter(deg_v, [idx16], ones16)

    pltpu.sync_copy(deg_v, out_hbm.at[pl.ds(wid * ACC_ROWS, ACC_ROWS)])


@functools.partial(
    pl.kernel,
    out_type=[
        jax.ShapeDtypeStruct((ACC_ROWS, D), jnp.float32),
        jax.ShapeDtypeStruct((ACC_ROWS, D), jnp.float32),
    ],
    mesh=_mesh(),
    scratch_types=[
        pltpu.VMEM_SHARED((ACC_ROWS, D), jnp.float32),
        [pltpu.VMEM((CHUNK, D), jnp.float32) for _ in range(2)],
        [pltpu.VMEM((CHUNK,), jnp.int32) for _ in range(2)],
        [pltpu.VMEM((CHUNK,), jnp.int32) for _ in range(2)],
        [pltpu.SemaphoreType.DMA for _ in range(2)],
        [pltpu.SemaphoreType.DMA for _ in range(2)],
        [pltpu.SemaphoreType.DMA for _ in range(2)],
        [pltpu.SemaphoreType.DMA for _ in range(2)],
    ],
    compiler_params=pltpu.CompilerParams(needs_layout_passes=False),
)
def _sc_gather_scatter(
    g_hbm, row_hbm, col_hbm, out0_hbm, out1_hbm, acc, bufs, cis, ris, gsems, ssems, csems, rsems
):
    c = lax.axis_index("c")
    s = lax.axis_index("s")
    wid = s * NC + c
    zeros16 = jnp.zeros((16,), jnp.float32)
    cbase, T = _tile_chunks(wid)
    r0 = s * RPT

    with jax.named_scope("zero_acc"):
        @pl.loop(0, CHUNK)
        def _zb(i):
            for j in range(D // 16):
                bufs[0][i, pl.ds(j * 16, 16)] = zeros16

        off = 0
        while off < RPT:
            take = min(CHUNK, RPT - off)
            pltpu.sync_copy(
                bufs[0].at[pl.ds(0, take)], acc.at[pl.ds(r0 + off, take)]
            )
            off += take
        plsc.subcore_barrier()

    def issue_cidx(t, k):
        pltpu.async_copy(
            col_hbm.at[pl.ds((cbase + t) * CHUNK, CHUNK)], cis[k], csems[k]
        )

    def wait_cidx(t, k):
        pltpu.make_async_copy(
            col_hbm.at[pl.ds((cbase + t) * CHUNK, CHUNK)], cis[k], csems[k]
        ).wait()

    def issue_ridx(t, k):
        pltpu.async_copy(
            row_hbm.at[pl.ds((cbase + t) * CHUNK, CHUNK)], ris[k], rsems[k]
        )

    def wait_ridx(t, k):
        pltpu.make_async_copy(
            row_hbm.at[pl.ds((cbase + t) * CHUNK, CHUNK)], ris[k], rsems[k]
        ).wait()

    def issue_gather(k):
        pltpu.async_copy(g_hbm.at[cis[k]], bufs[k], gsems[k])

    def wait_gather(k):
        pltpu.make_async_copy(g_hbm.at[cis[k]], bufs[k], gsems[k]).wait()

    def issue_scatter(k):
        pltpu.async_copy(bufs[k], acc.at[ris[k]], ssems[k], add=True)

    def wait_scatter(k):
        pltpu.make_async_copy(bufs[k], acc.at[ris[k]], ssems[k]).wait()

    with jax.named_scope("edge_loop"):
        issue_cidx(0, 0)

        @pl.loop(0, T // 2)
        def _steps(q):
            t0 = 2 * q
            for k in range(2):
                t = t0 + k
                k1 = 1 - k

                @pl.when(t >= 1)
                def _():
                    wait_ridx(t - 1, k1)
                    wait_gather(k1)
                    issue_scatter(k1)

                @pl.when(t + 1 < T)
                def _():
                    issue_cidx(t + 1, k1)

                @pl.when(t >= 2)
                def _():
                    wait_scatter(k)

                issue_ridx(t, k)
                wait_cidx(t, k)
                issue_gather(k)

        wait_ridx(T - 1, 1)
        wait_gather(1)
        issue_scatter(1)
        wait_scatter(0)
        wait_scatter(1)

    with jax.named_scope("writeout"):
        plsc.subcore_barrier()

        @pl.when(c == 0)
        def _w0():
            pltpu.sync_copy(acc.at[pl.ds(r0, RPT)], out0_hbm.at[pl.ds(r0, RPT)])

        @pl.when(c == 1)
        def _w1():
            pltpu.sync_copy(acc.at[pl.ds(r0, RPT)], out1_hbm.at[pl.ds(r0, RPT)])


RB2 = ACC_ROWS // 16


def _tc1_body(deg_ref, x_ref, w_ref, dinv_ref, g1_ref, out0_ref):
    deg = jnp.sum(deg_ref[...], axis=0)
    dinv = jnp.where(deg > 0, lax.rsqrt(jnp.where(deg > 0, deg, 1.0)), 0.0)
    d = dinv[:, None]
    dinv_ref[...] = d
    xv = x_ref[...]
    g1_ref[...] = d * xv
    out0_ref[...] = jnp.dot(xv, w_ref[...], preferred_element_type=jnp.float32)


def _tc1(degp, xp, w0):
    return pl.pallas_call(
        _tc1_body,
        grid=(1,),
        in_specs=[
            pl.BlockSpec((NW, ACC_ROWS), lambda i: (0, 0)),
            pl.BlockSpec((ACC_ROWS, D), lambda i: (0, 0)),
            pl.BlockSpec((D, D), lambda i: (0, 0)),
        ],
        out_specs=[
            pl.BlockSpec((ACC_ROWS, 1), lambda i: (0, 0)),
            pl.BlockSpec((ACC_ROWS, D), lambda i: (0, 0)),
            pl.BlockSpec((ACC_ROWS, D), lambda i: (0, 0)),
        ],
        out_shape=[
            jax.ShapeDtypeStruct((ACC_ROWS, 1), jnp.float32),
            jax.ShapeDtypeStruct((ACC_ROWS, D), jnp.float32),
            jax.ShapeDtypeStruct((ACC_ROWS, D), jnp.float32),
        ],
    )(degp, xp, w0)


def _tc2_body(a_ref, b_ref, dinv_ref, out0_ref, w_ref, out1_ref, g2_ref):
    d = dinv_ref[...]
    t = -d * (a_ref[...] + b_ref[...])
    out1_ref[...] = out0_ref[...] + jnp.dot(
        t, w_ref[...], preferred_element_type=jnp.float32
    )
    g2_ref[...] = d * t


def _tc2(s1a, s1b, dinv, out0, w1):
    row = pl.BlockSpec((RB2, D), lambda i: (i, 0))
    return pl.pallas_call(
        _tc2_body,
        grid=(ACC_ROWS // RB2,),
        in_specs=[
            row,
            row,
            pl.BlockSpec((RB2, 1), lambda i: (i, 0)),
            row,
            pl.BlockSpec((D, D), lambda i: (0, 0)),
        ],
        out_specs=[row, row],
        out_shape=[
            jax.ShapeDtypeStruct((ACC_ROWS, D), jnp.float32),
            jax.ShapeDtypeStruct((ACC_ROWS, D), jnp.float32),
        ],
    )(s1a, s1b, dinv, out0, w1)


def _tc3_body(a_ref, b_ref, dinv_ref, x_ref, out1_ref, w_ref, bias_ref, y_ref):
    d = dinv_ref[...]
    xv = x_ref[...]
    tx2 = -2.0 * d * (a_ref[...] + b_ref[...]) - xv
    o = (
        out1_ref[...]
        + jnp.dot(tx2, w_ref[...], preferred_element_type=jnp.float32)
        + bias_ref[...]
    )
    y_ref[...] = jnp.maximum(o + xv, 0.0)


def _tc3(s2a, s2b, dinv, xp, out1, w2, bias):
    row = pl.BlockSpec((RB2, D), lambda i: (i, 0))
    return pl.pallas_call(
        _tc3_body,
        grid=(ACC_ROWS // RB2,),
        in_specs=[
            row,
            row,
            pl.BlockSpec((RB2, 1), lambda i: (i, 0)),
            row,
            row,
            pl.BlockSpec((D, D), lambda i: (0, 0)),
            pl.BlockSpec((1, D), lambda i: (0, 0)),
        ],
        out_specs=row,
        out_shape=jax.ShapeDtypeStruct((ACC_ROWS, D), jnp.float32),
    )(s2a, s2b, dinv, xp, out1, w2, bias)


def kernel(x, edge_index, W, b):
    x = x.astype(jnp.float32)
    row = edge_index[0].astype(jnp.int32)
    col = edge_index[1].astype(jnp.int32)
    xp = jnp.concatenate([x, jnp.zeros((ACC_ROWS - N, D), jnp.float32)])

    degp = _sc_deg(row).reshape(NW, ACC_ROWS)
    dinv, g1, out0 = _tc1(degp, xp, W[0])
    s1a, s1b = _sc_gather_scatter(g1, row, col)
    out1, g2 = _tc2(s1a, s1b, dinv, out0, W[1])
    s2a, s2b = _sc_gather_scatter(g2, row, col)
    yp = _tc3(s2a, s2b, dinv, xp, out1, W[2], b.reshape(1, D))
    return yp[:N]


# speedup vs baseline: 1.1951x; 1.1951x over previous
import functools

import jax
import jax.numpy as jnp
from jax import lax
from jax.experimental import pallas as pl
from jax.experimental.pallas import tpu as pltpu
from jax.experimental.pallas import tpu_sc as plsc

N = 10000
D = 128
E = 320000
NC = 2
NS = 16
NW = NC * NS
CHUNK = 128
NCHUNKS = E // CHUNK
ACC_ROWS = N + 112
RPT = ACC_ROWS // NS


TMAIN = 78


def _tile_chunks(wid):
    return wid * TMAIN


def _mesh():
    return plsc.VectorSubcoreMesh(
        core_axis_name="c", subcore_axis_name="s", num_cores=NC, num_subcores=NS
    )


@functools.partial(
    pl.kernel,
    out_type=jax.ShapeDtypeStruct((NW * ACC_ROWS,), jnp.float32),
    mesh=_mesh(),
    scratch_types=[
        pltpu.VMEM((ACC_ROWS,), jnp.float32),
        pltpu.VMEM(((TMAIN + 1) * CHUNK,), jnp.int32),
        pltpu.SemaphoreType.DMA,
    ],
    compiler_params=pltpu.CompilerParams(needs_layout_passes=False),
)
def _sc_deg(row_hbm, out_hbm, deg_v, idx_v, sem):
    c = lax.axis_index("c")
    s = lax.axis_index("s")
    wid = s * NC + c
    zeros16 = jnp.zeros((16,), jnp.float32)
    ones16 = jnp.ones((16,), jnp.float32)
    cbase = _tile_chunks(wid)
    cnt = TMAIN + (wid < 4).astype(jnp.int32)

    idx_dma = pltpu.async_copy(
        row_hbm.at[pl.ds(cbase * CHUNK, TMAIN * CHUNK)],
        idx_v.at[pl.ds(0, TMAIN * CHUNK)],
        sem,
    )

    @pl.when(wid < 4)
    def _tail():
        pltpu.async_copy(
            row_hbm.at[pl.ds((NW * TMAIN + wid) * CHUNK, CHUNK)],
            idx_v.at[pl.ds(TMAIN * CHUNK, CHUNK)],
            sem,
        )

    @pl.loop(0, ACC_ROWS // 16)
    def _zero(i):
        deg_v[pl.ds(i * 16, 16)] = zeros16

    idx_dma.wait()

    @pl.when(wid < 4)
    def _tail_wait():
        pltpu.make_async_copy(
            row_hbm.at[pl.ds((NW * TMAIN + wid) * CHUNK, CHUNK)],
            idx_v.at[pl.ds(TMAIN * CHUNK, CHUNK)],
            sem,
        ).wait()

    @pl.loop(0, cnt * (CHUNK // 16))
    def _groups(i):
        idx16 = idx_v[pl.ds(i * 16, 16)]
        plsc.addupdate_scatter(deg_v, [idx16], ones16)

    pltpu.sync_copy(deg_v, out_hbm.at[pl.ds(wid * ACC_ROWS, ACC_ROWS)])


@functools.partial(
    pl.kernel,
    out_type=[
        jax.ShapeDtypeStruct((ACC_ROWS, D), jnp.float32),
        jax.ShapeDtypeStruct((ACC_ROWS, D), jnp.float32),
    ],
    mesh=_mesh(),
    scratch_types=[
        pltpu.VMEM_SHARED((ACC_ROWS, D), jnp.float32),
        [pltpu.VMEM((CHUNK, D), jnp.float32) for _ in range(3)],
        [pltpu.VMEM((CHUNK,), jnp.int32) for _ in range(3)],
        [pltpu.VMEM((CHUNK,), jnp.int32) for _ in range(3)],
        [pltpu.SemaphoreType.DMA for _ in range(3)],
        [pltpu.SemaphoreType.DMA for _ in range(3)],
        [pltpu.SemaphoreType.DMA for _ in range(3)],
        [pltpu.SemaphoreType.DMA for _ in range(3)],
    ],
    compiler_params=pltpu.CompilerParams(needs_layout_passes=False),
)
def _sc_gather_scatter(
    g_hbm, row_hbm, col_hbm, out0_hbm, out1_hbm, acc, bufs, cis, ris, gsems, ssems, csems, rsems
):
    c = lax.axis_index("c")
    s = lax.axis_index("s")
    wid = s * NC + c
    zeros16 = jnp.zeros((16,), jnp.float32)
    cbase = _tile_chunks(wid)
    T = TMAIN
    r0 = s * RPT

    with jax.named_scope("zero_acc"):
        @pl.loop(0, CHUNK)
        def _zb(i):
            for j in range(D // 16):
                bufs[0][i, pl.ds(j * 16, 16)] = zeros16

        off = 0
        while off < RPT:
            take = min(CHUNK, RPT - off)
            pltpu.sync_copy(
                bufs[0].at[pl.ds(0, take)], acc.at[pl.ds(r0 + off, take)]
            )
            off += take
        plsc.subcore_barrier()

    def issue_cidx(t, k):
        pltpu.async_copy(
            col_hbm.at[pl.ds((cbase + t) * CHUNK, CHUNK)], cis[k], csems[k]
        )

    def wait_cidx(t, k):
        pltpu.make_async_copy(
            col_hbm.at[pl.ds((cbase + t) * CHUNK, CHUNK)], cis[k], csems[k]
        ).wait()

    def issue_ridx(t, k):
        pltpu.async_copy(
            row_hbm.at[pl.ds((cbase + t) * CHUNK, CHUNK)], ris[k], rsems[k]
        )

    def wait_ridx(t, k):
        pltpu.make_async_copy(
            row_hbm.at[pl.ds((cbase + t) * CHUNK, CHUNK)], ris[k], rsems[k]
        ).wait()

    def issue_gather(k):
        pltpu.async_copy(g_hbm.at[cis[k]], bufs[k], gsems[k])

    def wait_gather(k):
        pltpu.make_async_copy(g_hbm.at[cis[k]], bufs[k], gsems[k]).wait()

    def issue_scatter(k):
        pltpu.async_copy(bufs[k], acc.at[ris[k]], ssems[k], add=True)

    def wait_scatter(k):
        pltpu.make_async_copy(bufs[k], acc.at[ris[k]], ssems[k]).wait()

    with jax.named_scope("edge_loop"):
        issue_cidx(0, 0)
        issue_cidx(1, 1)
        issue_ridx(0, 0)
        wait_cidx(0, 0)
        issue_gather(0)

        @pl.loop(0, T // 3)
        def _steps(q):
            t0 = 3 * q
            for k in range(3):
                t = t0 + k
                kp = (k + 2) % 3
                kn = (k + 1) % 3

                @pl.when(t >= 1)
                def _():
                    wait_ridx(t - 1, kp)
                    wait_gather(kp)
                    issue_scatter(kp)

                @pl.when(t + 2 < T)
                def _():
                    issue_cidx(t + 2, kp)

                @pl.when(t + 1 < T)
                def _():
                    @pl.when(t >= 2)
                    def _():
                        wait_scatter(kn)

                    issue_ridx(t + 1, kn)
                    wait_cidx(t + 1, kn)
                    issue_gather(kn)

        kl = (T - 1) % 3
        wait_ridx(T - 1, kl)
        wait_gather(kl)
        issue_scatter(kl)
        wait_scatter(0)
        wait_scatter(1)
        wait_scatter(2)

        @pl.when(wid < 4)
        def _leftover():
            tl = NW * TMAIN + wid - cbase
            issue_cidx(tl, 0)
            issue_ridx(tl, 0)
            wait_cidx(tl, 0)
            wait_ridx(tl, 0)
            issue_gather(0)
            wait_gather(0)
            issue_scatter(0)
            wait_scatter(0)

    with jax.named_scope("writeout"):
        plsc.subcore_barrier()

        @pl.when(c == 0)
        def _w0():
            pltpu.sync_copy(acc.at[pl.ds(r0, RPT)], out0_hbm.at[pl.ds(r0, RPT)])

        @pl.when(c == 1)
        def _w1():
            pltpu.sync_copy(acc.at[pl.ds(r0, RPT)], out1_hbm.at[pl.ds(r0, RPT)])


RB2 = ACC_ROWS // 16


def _tc1_body(deg_ref, x_ref, w_ref, dinv_ref, g1_ref, out0_ref):
    deg = jnp.sum(deg_ref[...], axis=0)
    dinv = jnp.where(deg > 0, lax.rsqrt(jnp.where(deg > 0, deg, 1.0)), 0.0)
    d = dinv[:, None]
    dinv_ref[...] = d
    xv = x_ref[...]
    g1_ref[...] = d * xv
    out0_ref[...] = jnp.dot(xv, w_ref[...], preferred_element_type=jnp.float32)


def _tc1(degp, xp, w0):
    return pl.pallas_call(
        _tc1_body,
        grid=(1,),
        in_specs=[
            pl.BlockSpec((NW, ACC_ROWS), lambda i: (0, 0)),
            pl.BlockSpec((ACC_ROWS, D), lambda i: (0, 0)),
            pl.BlockSpec((D, D), lambda i: (0, 0)),
        ],
        out_specs=[
            pl.BlockSpec((ACC_ROWS, 1), lambda i: (0, 0)),
            pl.BlockSpec((ACC_ROWS, D), lambda i: (0, 0)),
            pl.BlockSpec((ACC_ROWS, D), lambda i: (0, 0)),
        ],
        out_shape=[
            jax.ShapeDtypeStruct((ACC_ROWS, 1), jnp.float32),
            jax.ShapeDtypeStruct((ACC_ROWS, D), jnp.float32),
            jax.ShapeDtypeStruct((ACC_ROWS, D), jnp.float32),
        ],
    )(degp, xp, w0)


def _tc2_body(a_ref, b_ref, dinv_ref, out0_ref, w_ref, out1_ref, g2_ref):
    d = dinv_ref[...]
    t = -d * (a_ref[...] + b_ref[...])
    out1_ref[...] = out0_ref[...] + jnp.dot(
        t, w_ref[...], preferred_element_type=jnp.float32
    )
    g2_ref[...] = d * t


def _tc2(s1a, s1b, dinv, out0, w1):
    row = pl.BlockSpec((RB2, D), lambda i: (i, 0))
    return pl.pallas_call(
        _tc2_body,
        grid=(ACC_ROWS // RB2,),
        in_specs=[
            row,
            row,
            pl.BlockSpec((RB2, 1), lambda i: (i, 0)),
            row,
            pl.BlockSpec((D, D), lambda i: (0, 0)),
        ],
        out_specs=[row, row],
        out_shape=[
            jax.ShapeDtypeStruct((ACC_ROWS, D), jnp.float32),
            jax.ShapeDtypeStruct((ACC_ROWS, D), jnp.float32),
        ],
    )(s1a, s1b, dinv, out0, w1)


def _tc3_body(a_ref, b_ref, dinv_ref, x_ref, out1_ref, w_ref, bias_ref, y_ref):
    d = dinv_ref[...]
    xv = x_ref[...]
    tx2 = -2.0 * d * (a_ref[...] + b_ref[...]) - xv
    o = (
        out1_ref[...]
        + jnp.dot(tx2, w_ref[...], preferred_element_type=jnp.float32)
        + bias_ref[...]
    )
    y_ref[...] = jnp.maximum(o + xv, 0.0)


def _tc3(s2a, s2b, dinv, xp, out1, w2, bias):
    row = pl.BlockSpec((RB2, D), lambda i: (i, 0))
    return pl.pallas_call(
        _tc3_body,
        grid=(ACC_ROWS // RB2,),
        in_specs=[
            row,
            row,
            pl.BlockSpec((RB2, 1), lambda i: (i, 0)),
            row,
            row,
            pl.BlockSpec((D, D), lambda i: (0, 0)),
            pl.BlockSpec((1, D), lambda i: (0, 0)),
        ],
        out_specs=row,
        out_shape=jax.ShapeDtypeStruct((ACC_ROWS, D), jnp.float32),
    )(s2a, s2b, dinv, xp, out1, w2, bias)


def kernel(x, edge_index, W, b):
    x = x.astype(jnp.float32)
    row = edge_index[0].astype(jnp.int32)
    col = edge_index[1].astype(jnp.int32)
    xp = jnp.concatenate([x, jnp.zeros((ACC_ROWS - N, D), jnp.float32)])

    degp = _sc_deg(row).reshape(NW, ACC_ROWS)
    dinv, g1, out0 = _tc1(degp, xp, W[0])
    s1a, s1b = _sc_gather_scatter(g1, row, col)
    out1, g2 = _tc2(s1a, s1b, dinv, out0, W[1])
    s2a, s2b = _sc_gather_scatter(g2, row, col)
    yp = _tc3(s2a, s2b, dinv, xp, out1, W[2], b.reshape(1, D))
    return yp[:N]
